# baseline (speedup 1.0000x reference)
import functools

import jax
import jax.numpy as jnp
from jax import lax
from jax.experimental import pallas as pl
from jax.experimental.pallas import tpu as pltpu
from jax.experimental.pallas import tpu_sc as plsc

EMBED = 32
BATCH = 16384
NUM_CORES = 2
NUM_SUBCORES = 16
NUM_WORKERS = NUM_CORES * NUM_SUBCORES
B_PER_W = BATCH // NUM_WORKERS


def _make_gather():
    mesh = plsc.VectorSubcoreMesh(core_axis_name="c", subcore_axis_name="s")

    @functools.partial(
        pl.kernel,
        mesh=mesh,
        out_type=jax.ShapeDtypeStruct((BATCH, EMBED), jnp.float32),
        scratch_types=[
            pltpu.VMEM((B_PER_W,), jnp.int32),
            pltpu.VMEM((8, EMBED), jnp.float32),
            pltpu.VMEM((B_PER_W, EMBED), jnp.float32),
            pltpu.SemaphoreType.DMA,
        ],
        compiler_params=pltpu.CompilerParams(use_tc_tiling_on_sc=True),
    )
    def gather_kernel(table_hbm, ids_hbm, out_hbm, idx_v, tile_v, out_v, sem):
        wid = lax.axis_index("s") * NUM_CORES + lax.axis_index("c")
        base = wid * B_PER_W
        pltpu.sync_copy(ids_hbm.at[pl.ds(base, B_PER_W)], idx_v)

        def step(vi, carry):
            vec = idx_v[pl.ds(vi * 16, 16)]
            for j in range(16):
                r = vec[j]
                g = (r // 8) * 8
                pltpu.async_copy(table_hbm.at[pl.ds(g, 8)], tile_v, sem).wait()
                rr = r - g
                i = vi * 16 + j
                out_v[i, pl.ds(0, 16)] = tile_v[rr, pl.ds(0, 16)]
                out_v[i, pl.ds(16, 16)] = tile_v[rr, pl.ds(16, 16)]
            return carry

        lax.fori_loop(0, B_PER_W // 16, step, 0)
        pltpu.sync_copy(out_v, out_hbm.at[pl.ds(base, B_PER_W)])

    return gather_kernel


_gather = _make_gather()


def kernel(table, ids):
    return _gather(table, ids)


# speedup vs baseline: 1.7826x; 1.7826x over previous
import functools

import jax
import jax.numpy as jnp
from jax import lax
from jax.experimental import pallas as pl
from jax.experimental.pallas import tpu as pltpu
from jax.experimental.pallas import tpu_sc as plsc

EMBED = 32
BATCH = 16384
NUM_CORES = 2
NUM_SUBCORES = 16
NUM_WORKERS = NUM_CORES * NUM_SUBCORES
B_PER_W = BATCH // NUM_WORKERS
G = 16
NGROUPS = B_PER_W // G


def _make_gather():
    mesh = plsc.VectorSubcoreMesh(core_axis_name="c", subcore_axis_name="s")

    @functools.partial(
        pl.kernel,
        mesh=mesh,
        out_type=jax.ShapeDtypeStruct((BATCH, EMBED), jnp.float32),
        scratch_types=[
            pltpu.VMEM((B_PER_W,), jnp.int32),
            pltpu.VMEM((G, 8, EMBED), jnp.float32),
            pltpu.VMEM((G, 8, EMBED), jnp.float32),
            pltpu.VMEM((B_PER_W, EMBED), jnp.float32),
            pltpu.SemaphoreType.DMA,
            pltpu.SemaphoreType.DMA,
        ],
        compiler_params=pltpu.CompilerParams(use_tc_tiling_on_sc=True),
    )
    def gather_kernel(table_hbm, ids_hbm, out_hbm, idx_v, bufa_v, bufb_v, out_v,
                      sema, semb):
        wid = lax.axis_index("s") * NUM_CORES + lax.axis_index("c")
        base = wid * B_PER_W
        pltpu.sync_copy(ids_hbm.at[pl.ds(base, B_PER_W)], idx_v)

        def issue(g, buf, sem):
            vec = idx_v[pl.ds(g * G, G)]
            for j in range(G):
                grp = (vec[j] // 8) * 8
                pltpu.async_copy(table_hbm.at[pl.ds(grp, 8)], buf.at[j], sem)

        def drain_extract(g, buf, sem):
            for j in range(G):
                pltpu.make_async_copy(table_hbm.at[pl.ds(0, 8)], buf.at[j],
                                      sem).wait()
            vec = idx_v[pl.ds(g * G, G)]
            for j in range(G):
                r = vec[j]
                rr = r - (r // 8) * 8
                i = g * G + j
                out_v[i, pl.ds(0, 16)] = buf[j, rr, pl.ds(0, 16)]
                out_v[i, pl.ds(16, 16)] = buf[j, rr, pl.ds(16, 16)]

        issue(0, bufa_v, sema)

        def pair(vi, carry):
            ga = vi * 2
            issue(ga + 1, bufb_v, semb)
            drain_extract(ga, bufa_v, sema)

            @pl.when(vi < NGROUPS // 2 - 1)
            def _():
                issue(ga + 2, bufa_v, sema)

            drain_extract(ga + 1, bufb_v, semb)
            return carry

        lax.fori_loop(0, NGROUPS // 2, pair, 0)
        pltpu.sync_copy(out_v, out_hbm.at[pl.ds(base, B_PER_W)])

    return gather_kernel


_gather = _make_gather()


def kernel(table, ids):
    return _gather(table, ids)


# speedup vs baseline: 1.9514x; 1.0947x over previous
import functools

import jax
import jax.numpy as jnp
from jax import lax
from jax.experimental import pallas as pl
from jax.experimental.pallas import tpu as pltpu
from jax.experimental.pallas import tpu_sc as plsc

EMBED = 32
BATCH = 16384
NUM_CORES = 2
NUM_SUBCORES = 16
NUM_WORKERS = NUM_CORES * NUM_SUBCORES
B_PER_W = BATCH // NUM_WORKERS
G = 16
NGROUPS = B_PER_W // G


def _make_gather():
    mesh = plsc.VectorSubcoreMesh(core_axis_name="c", subcore_axis_name="s")

    @functools.partial(
        pl.kernel,
        mesh=mesh,
        out_type=jax.ShapeDtypeStruct((BATCH, EMBED), jnp.float32),
        scratch_types=[
            pltpu.VMEM((B_PER_W,), jnp.int32),
            pltpu.VMEM((B_PER_W, EMBED), jnp.float32),
            pltpu.SemaphoreType.DMA,
        ],
        compiler_params=pltpu.CompilerParams(use_tc_tiling_on_sc=True),
    )
    def gather_kernel(table_hbm, ids_hbm, out_hbm, idx_v, out_v, sem):
        wid = lax.axis_index("s") * NUM_CORES + lax.axis_index("c")
        base = wid * B_PER_W
        pltpu.sync_copy(ids_hbm.at[pl.ds(base, B_PER_W)], idx_v)

        def issue_group(g, carry):
            vec = idx_v[pl.ds(g * G, G)]
            for j in range(G):
                pltpu.async_copy(table_hbm.at[pl.ds(vec[j], 1)],
                                 out_v.at[pl.ds(g * G + j, 1)], sem)
            return carry

        lax.fori_loop(0, NGROUPS, issue_group, 0)
        pltpu.make_async_copy(table_hbm.at[pl.ds(0, B_PER_W)], out_v,
                              sem).wait()
        pltpu.sync_copy(out_v, out_hbm.at[pl.ds(base, B_PER_W)])

    return gather_kernel


_gather = _make_gather()


def kernel(table, ids):
    return _gather(table, ids)
